# baseline (speedup 1.0000x reference)
import functools

import jax
import jax.numpy as jnp
from jax import lax
from jax.experimental import pallas as pl
from jax.experimental.pallas import tpu as pltpu
from jax.experimental.pallas import tpu_sc as plsc

_B, _S, _D = 4, 4096, 1024
_ROWS = _B * _S
_NC, _NS = 2, 16
_NW = _NC * _NS
_RPW = _ROWS // _NW
_CH = 64
_NCH = _RPW // _CH
_WPS = _S // _RPW


@functools.partial(
    pl.kernel,
    mesh=plsc.VectorSubcoreMesh(
        core_axis_name="c", subcore_axis_name="s",
        num_cores=_NC, num_subcores=_NS),
    out_type=jax.ShapeDtypeStruct((_ROWS, _D), jnp.float32),
    scratch_types=[
        pltpu.VMEM((_NCH, _CH), jnp.int32),
        pltpu.VMEM((_CH, _D), jnp.float32),
        pltpu.SemaphoreType.DMA,
    ],
)
def _emb(idx_hbm, w_hbm, out_hbm, idx_v, buf, sem):
    wid = lax.axis_index("s") * _NC + lax.axis_index("c")
    seq_blk = lax.rem(wid, _WPS)
    pltpu.sync_copy(idx_hbm.at[seq_blk], idx_v)
    base = wid * _RPW

    def chunk(c, carry):
        pltpu.async_copy(w_hbm.at[idx_v.at[c]], buf, sem).wait()
        pltpu.sync_copy(buf, out_hbm.at[pl.ds(base + c * _CH, _CH)])
        return carry

    lax.fori_loop(0, _NCH, chunk, 0)


def kernel(x, idx, W):
    idx3 = idx.reshape(_WPS, _NCH, _CH)
    out = _emb(idx3, W)
    return out.reshape(_B, _S, _D)


# speedup vs baseline: 8.5490x; 8.5490x over previous
import functools

import jax
import jax.numpy as jnp
from jax import lax
from jax.experimental import pallas as pl
from jax.experimental.pallas import tpu as pltpu
from jax.experimental.pallas import tpu_sc as plsc

_B, _S, _D = 4, 4096, 1024
_NC, _NS = 2, 16
_NW = _NC * _NS
_SPW = _S // _NW
_CH = 32
_NCH = _SPW // _CH
_L = 16
_REP = 512


@functools.partial(
    pl.kernel,
    mesh=plsc.VectorSubcoreMesh(
        core_axis_name="c", subcore_axis_name="s",
        num_cores=_NC, num_subcores=_NS),
    out_type=jax.ShapeDtypeStruct((_B, _S, _D), jnp.float32),
    scratch_types=[
        pltpu.VMEM((_SPW,), jnp.int32),
        pltpu.VMEM((_NCH, _CH), jnp.int32),
        pltpu.VMEM((2, _CH, _D), jnp.float32),
        pltpu.SemaphoreType.DMA,
        pltpu.SemaphoreType.DMA,
    ],
)
def _emb(idx_hbm, w_hbm, out_hbm, idx_l, sidx, bufs, gsem, wsem):
    wid = lax.axis_index("s") * _NC + lax.axis_index("c")
    seq0 = wid * _SPW
    pltpu.sync_copy(idx_hbm.at[wid], idx_l)
    lanes = lax.iota(jnp.int32, _L)

    for c in range(_NCH):
        for h in range(_CH // _L):
            off = c * _CH + h * _L
            iv = idx_l[pl.ds(off, _L)]
            spread = lax.rem(seq0 + off + lanes, _REP)
            sidx[c, pl.ds(h * _L, _L)] = iv + 2 * spread

    def out_slice(c, b):
        return out_hbm.at[b, pl.ds(seq0 + c * _CH, _CH)]

    def chunk(c, carry):
        buf = bufs.at[lax.rem(c, 2)]

        @pl.when(c >= 2)
        def _drain_writes():
            for b in range(_B):
                pltpu.make_async_copy(buf, out_slice(c - 2, b), wsem).wait()

        pltpu.async_copy(w_hbm.at[sidx.at[c]], buf, gsem).wait()
        for b in range(_B):
            pltpu.async_copy(buf, out_slice(c, b), wsem)
        return carry

    lax.fori_loop(0, _NCH, chunk, 0)
    for c in (_NCH - 2, _NCH - 1):
        buf = bufs.at[lax.rem(c, 2)]
        for b in range(_B):
            pltpu.make_async_copy(buf, out_slice(c, b), wsem).wait()


def kernel(x, idx, W):
    idx2 = idx.reshape(_NW, _SPW)
    w_rep = jnp.tile(W, (_REP, 1))
    out = _emb(idx2, w_rep)
    return out


# speedup vs baseline: 8.6136x; 1.0076x over previous
import functools

import jax
import jax.numpy as jnp
from jax import lax
from jax.experimental import pallas as pl
from jax.experimental.pallas import tpu as pltpu
from jax.experimental.pallas import tpu_sc as plsc

_B, _S, _D = 4, 4096, 1024
_NC, _NS = 2, 16
_NW = _NC * _NS
_SPW = _S // _NW
_CH = 32
_NCH = _SPW // _CH
_NB = 3
_L = 16
_REP = 512


@functools.partial(
    pl.kernel,
    mesh=plsc.VectorSubcoreMesh(
        core_axis_name="c", subcore_axis_name="s",
        num_cores=_NC, num_subcores=_NS),
    out_type=jax.ShapeDtypeStruct((_B, _S, _D), jnp.float32),
    scratch_types=[
        pltpu.VMEM((_SPW,), jnp.int32),
        pltpu.VMEM((_NCH, _CH), jnp.int32),
        pltpu.VMEM((_NB, _CH, _D), jnp.float32),
        pltpu.SemaphoreType.DMA,
        pltpu.SemaphoreType.DMA,
    ],
)
def _emb(idx_hbm, w_hbm, out_hbm, idx_l, sidx, bufs, gsem, wsem):
    wid = lax.axis_index("s") * _NC + lax.axis_index("c")
    seq0 = wid * _SPW
    pltpu.sync_copy(idx_hbm.at[wid], idx_l)
    lanes = lax.iota(jnp.int32, _L)

    for c in range(_NCH):
        for h in range(_CH // _L):
            off = c * _CH + h * _L
            iv = idx_l[pl.ds(off, _L)]
            spread = lax.rem(seq0 + off + lanes, _REP)
            sidx[c, pl.ds(h * _L, _L)] = iv + 2 * spread

    def out_slice(c, b):
        return out_hbm.at[b, pl.ds(seq0 + c * _CH, _CH)]

    def gather(c):
        return pltpu.async_copy(w_hbm.at[sidx.at[c]], bufs.at[c % _NB], gsem)

    def wait_writes(c):
        for b in range(_B):
            pltpu.make_async_copy(
                bufs.at[c % _NB], out_slice(c, b), wsem
            ).wait()

    gather(0)
    for c in range(_NCH):
        pltpu.make_async_copy(
            w_hbm.at[sidx.at[c]], bufs.at[c % _NB], gsem
        ).wait()
        if c + 1 < _NCH:
            if c + 1 >= _NB:
                wait_writes(c + 1 - _NB)
            gather(c + 1)
        for b in range(_B):
            pltpu.async_copy(bufs.at[c % _NB], out_slice(c, b), wsem)
    for c in range(max(0, _NCH - _NB), _NCH):
        wait_writes(c)


def kernel(x, idx, W):
    idx2 = idx.reshape(_NW, _SPW)
    w_rep = jnp.tile(W, (_REP, 1))
    return _emb(idx2, w_rep)
